# Initial kernel scaffold; baseline (speedup 1.0000x reference)
import jax
import jax.numpy as jnp
from jax.experimental import pallas as pl


def kernel(x, edge_index, edge_weight, W1, b1, W2, b2):
    raise NotImplementedError("write your pallas kernel here")


# speedup vs baseline: 35.1390x; 35.1390x over previous
import functools

import jax
import jax.numpy as jnp
from jax import lax
from jax.experimental import pallas as pl
from jax.experimental.pallas import tpu as pltpu
from jax.experimental.pallas import tpu_sc as plsc

N = 10000
E = 320000
D_IN = 128
D_HID = 16

NC = 2
NS = 16
NW = NC * NS
BLK = 128
BPT = 79
EPT = BPT * BLK
EP = NW * EPT
NP = 10240
STRIPE = NP // NS

_mesh = plsc.VectorSubcoreMesh(core_axis_name="c", subcore_axis_name="s")
_f32 = jnp.float32
_sc_params = pltpu.CompilerParams(use_tc_tiling_on_sc=False)


@functools.partial(
    pl.kernel,
    out_type=jax.ShapeDtypeStruct((NC * NP,), _f32),
    mesh=_mesh,
    scratch_types=[
        pltpu.VMEM((BPT, BLK), jnp.int32),
        pltpu.VMEM((BPT, BLK), _f32),
        pltpu.VMEM((BLK,), _f32),
        pltpu.VMEM_SHARED((NP,), _f32),
    ],
    compiler_params=_sc_params,
)
def _sc_degree(col_hbm, ew_hbm, out_hbm, colv, ewv, zbuf, acc):
    c = lax.axis_index("c")
    s = lax.axis_index("s")
    wid = s * NC + c
    for v in range(BLK // 16):
        zbuf[pl.ds(v * 16, 16)] = jnp.zeros((16,), _f32)
    for k in range(STRIPE // BLK):
        pltpu.sync_copy(zbuf, acc.at[pl.ds(s * STRIPE + k * BLK, BLK)])
    plsc.subcore_barrier()
    pltpu.sync_copy(col_hbm.at[wid], colv)
    pltpu.sync_copy(ew_hbm.at[wid], ewv)

    def blk(b, carry):
        pltpu.sync_copy(ewv.at[b], acc.at[colv.at[b]], add=True)
        return carry

    lax.fori_loop(0, BPT, blk, 0)
    plsc.subcore_barrier()
    pltpu.sync_copy(acc.at[pl.ds(s * STRIPE, STRIPE)],
                    out_hbm.at[pl.ds(c * NP + s * STRIPE, STRIPE)])


@functools.partial(
    pl.kernel,
    out_type=jax.ShapeDtypeStruct((NC * NP, D_HID), _f32),
    mesh=_mesh,
    scratch_types=[
        pltpu.VMEM((BPT, BLK), jnp.int32),
        pltpu.VMEM((BPT, BLK), jnp.int32),
        pltpu.VMEM((BPT, BLK), _f32),
        pltpu.VMEM((BLK, D_HID), _f32),
        pltpu.VMEM_SHARED((NP, D_HID), _f32),
        pltpu.SemaphoreType.DMA,
    ],
    compiler_params=_sc_params,
)
def _sc_msgpass16(row_hbm, col_hbm, ew_hbm, hp_hbm, out_hbm,
                  rowv, colv, ewv, gbuf, acc, sem):
    c = lax.axis_index("c")
    s = lax.axis_index("s")
    wid = s * NC + c
    zero16 = jnp.zeros((16,), _f32)

    def zi(j, carry):
        gbuf[j] = zero16
        return carry

    lax.fori_loop(0, BLK, zi, 0)
    for k in range(STRIPE // BLK):
        pltpu.sync_copy(gbuf, acc.at[pl.ds(s * STRIPE + k * BLK, BLK)])
    plsc.subcore_barrier()
    pltpu.sync_copy(row_hbm.at[wid], rowv)
    pltpu.sync_copy(col_hbm.at[wid], colv)
    pltpu.sync_copy(ew_hbm.at[wid], ewv)

    def blk(b, carry):
        pltpu.async_copy(hp_hbm.at[rowv.at[b]], gbuf, sem).wait()

        def scale16(t, cc):
            ew16 = ewv[b, pl.ds(t * 16, 16)]
            for j in range(16):
                r = t * 16 + j
                gbuf[r] = gbuf[r] * ew16[j]
            return cc

        lax.fori_loop(0, BLK // 16, scale16, 0)
        pltpu.sync_copy(gbuf, acc.at[colv.at[b]], add=True)
        return carry

    lax.fori_loop(0, BPT, blk, 0)
    plsc.subcore_barrier()
    pltpu.sync_copy(acc.at[pl.ds(s * STRIPE, STRIPE)],
                    out_hbm.at[pl.ds(c * NP + s * STRIPE, STRIPE)])


@functools.partial(
    pl.kernel,
    out_type=jax.ShapeDtypeStruct((NC * NP,), _f32),
    mesh=_mesh,
    scratch_types=[
        pltpu.VMEM((BPT, BLK), jnp.int32),
        pltpu.VMEM((BPT, BLK), jnp.int32),
        pltpu.VMEM((BPT, BLK), _f32),
        pltpu.VMEM((BLK,), _f32),
        pltpu.VMEM_SHARED((NP,), _f32),
        pltpu.SemaphoreType.DMA,
    ],
    compiler_params=_sc_params,
)
def _sc_msgpass1(row_hbm, col_hbm, ew_hbm, hp_hbm, out_hbm,
                 rowv, colv, ewv, gbuf, acc, sem):
    c = lax.axis_index("c")
    s = lax.axis_index("s")
    wid = s * NC + c
    for v in range(BLK // 16):
        gbuf[pl.ds(v * 16, 16)] = jnp.zeros((16,), _f32)
    for k in range(STRIPE // BLK):
        pltpu.sync_copy(gbuf, acc.at[pl.ds(s * STRIPE + k * BLK, BLK)])
    plsc.subcore_barrier()
    pltpu.sync_copy(row_hbm.at[wid], rowv)
    pltpu.sync_copy(col_hbm.at[wid], colv)
    pltpu.sync_copy(ew_hbm.at[wid], ewv)

    def blk(b, carry):
        pltpu.async_copy(hp_hbm.at[rowv.at[b]], gbuf, sem).wait()
        for v in range(BLK // 16):
            sl = pl.ds(v * 16, 16)
            gbuf[sl] = gbuf[sl] * ewv[b, sl]
        pltpu.sync_copy(gbuf, acc.at[colv.at[b]], add=True)
        return carry

    lax.fori_loop(0, BPT, blk, 0)
    plsc.subcore_barrier()
    pltpu.sync_copy(acc.at[pl.ds(s * STRIPE, STRIPE)],
                    out_hbm.at[pl.ds(c * NP + s * STRIPE, STRIPE)])


def _dense1_body(d0, d1, x, w1, dis_o, h1p_o):
    dis = lax.rsqrt(d0[...] + d1[...] + 1.0)
    dis_o[...] = dis
    h1 = jnp.dot(x[...], w1[...], preferred_element_type=_f32)
    h1p_o[...] = h1 * dis


def _dense2_body(a0, a1, h1p, dis, b1, w2, h2p_o):
    y1 = jnp.maximum(dis[...] * (a0[...] + a1[...] + h1p[...]) + b1[...], 0.0)
    h2 = jnp.dot(y1, w2[...], preferred_element_type=_f32)
    h2p_o[...] = h2 * dis[...]


def _final_body(q0, q1, h2p, dis, b2, out_o):
    out_o[...] = dis[...] * (q0[...] + q1[...] + h2p[...]) + b2[...]


_dense1 = pl.pallas_call(
    _dense1_body,
    out_shape=[jax.ShapeDtypeStruct((NP, 1), _f32),
               jax.ShapeDtypeStruct((NP, D_HID), _f32)],
)

_dense2 = pl.pallas_call(
    _dense2_body,
    out_shape=jax.ShapeDtypeStruct((NP, 1), _f32),
)

_final = pl.pallas_call(
    _final_body,
    out_shape=jax.ShapeDtypeStruct((NP, 1), _f32),
)


def kernel(x, edge_index, edge_weight, W1, b1, W2, b2):
    ei = edge_index.astype(jnp.int32)
    row = jnp.pad(ei[0], (0, EP - E)).reshape(NW, BPT, BLK)
    col = jnp.pad(ei[1], (0, EP - E)).reshape(NW, BPT, BLK)
    ew = jnp.pad(edge_weight.astype(_f32), (0, EP - E)).reshape(NW, BPT, BLK)
    x_p = jnp.pad(x.astype(_f32), ((0, NP - N), (0, 0)))

    degp = _sc_degree(col, ew)
    dis, h1p = _dense1(degp[:NP].reshape(NP, 1), degp[NP:].reshape(NP, 1),
                       x_p, W1.astype(_f32))
    accp = _sc_msgpass16(row, col, ew, h1p)
    h2p = _dense2(accp[:NP], accp[NP:], h1p, dis,
                  b1.astype(_f32).reshape(1, D_HID), W2.astype(_f32))
    qp = _sc_msgpass1(row, col, ew, h2p.reshape(NP))
    out = _final(qp[:NP].reshape(NP, 1), qp[NP:].reshape(NP, 1),
                 h2p, dis, b2.astype(_f32).reshape(1, 1))
    return out[:N]


# speedup vs baseline: 39.3139x; 1.1188x over previous
import functools

import jax
import jax.numpy as jnp
from jax import lax
from jax.experimental import pallas as pl
from jax.experimental.pallas import tpu as pltpu
from jax.experimental.pallas import tpu_sc as plsc

N = 10000
E = 320000
D_IN = 128
D_HID = 16

NC = 2
NS = 16
NW = NC * NS
BLK = 128
GRP = 8
BPT = 80
NG = BPT // GRP
EPT = BPT * BLK
EP = NW * EPT
NP = 10240
STRIPE = NP // NS

_mesh = plsc.VectorSubcoreMesh(core_axis_name="c", subcore_axis_name="s")
_f32 = jnp.float32
_sc_params = pltpu.CompilerParams(use_tc_tiling_on_sc=False)


@functools.partial(
    pl.kernel,
    out_type=jax.ShapeDtypeStruct((NC * NP,), _f32),
    mesh=_mesh,
    scratch_types=[
        pltpu.VMEM((BPT, BLK), jnp.int32),
        pltpu.VMEM((BPT, BLK), _f32),
        pltpu.VMEM((BLK,), _f32),
        pltpu.VMEM_SHARED((NP,), _f32),
        pltpu.SemaphoreType.DMA,
    ],
    compiler_params=_sc_params,
)
def _sc_degree(col_hbm, ew_hbm, out_hbm, colv, ewv, zbuf, acc, sem):
    c = lax.axis_index("c")
    s = lax.axis_index("s")
    wid = s * NC + c
    for v in range(BLK // 16):
        zbuf[pl.ds(v * 16, 16)] = jnp.zeros((16,), _f32)
    for k in range(STRIPE // BLK):
        pltpu.async_copy(zbuf, acc.at[pl.ds(s * STRIPE + k * BLK, BLK)], sem)
    for k in range(STRIPE // BLK):
        pltpu.make_async_copy(zbuf, acc.at[pl.ds(s * STRIPE, BLK)], sem).wait()
    plsc.subcore_barrier()
    pltpu.sync_copy(col_hbm.at[wid], colv)
    pltpu.sync_copy(ew_hbm.at[wid], ewv)

    def grp(g, carry):
        base = g * GRP
        for j in range(GRP):
            pltpu.async_copy(ewv.at[base + j], acc.at[colv.at[base + j]],
                             sem, add=True)
        for j in range(GRP):
            pltpu.make_async_copy(ewv.at[base + j],
                                  acc.at[colv.at[base + j]], sem).wait()
        return carry

    lax.fori_loop(0, NG, grp, 0)
    plsc.subcore_barrier()
    pltpu.sync_copy(acc.at[pl.ds(s * STRIPE, STRIPE)],
                    out_hbm.at[pl.ds(c * NP + s * STRIPE, STRIPE)])


@functools.partial(
    pl.kernel,
    out_type=jax.ShapeDtypeStruct((NC * NP, D_HID), _f32),
    mesh=_mesh,
    scratch_types=[
        pltpu.VMEM((BPT, BLK), jnp.int32),
        pltpu.VMEM((BPT, BLK), jnp.int32),
        pltpu.VMEM((BPT, BLK), _f32),
        pltpu.VMEM((GRP, BLK, D_HID), _f32),
        pltpu.VMEM_SHARED((NP, D_HID), _f32),
        pltpu.SemaphoreType.DMA,
        pltpu.SemaphoreType.DMA,
    ],
    compiler_params=_sc_params,
)
def _sc_msgpass16(row_hbm, col_hbm, ew_hbm, hp_hbm, out_hbm,
                  rowv, colv, ewv, bufs, acc, semg, sema):
    c = lax.axis_index("c")
    s = lax.axis_index("s")
    wid = s * NC + c
    z = bufs.at[0]
    zero16 = jnp.zeros((16,), _f32)

    def zi(j, carry):
        z[j] = zero16
        return carry

    lax.fori_loop(0, BLK, zi, 0)
    for k in range(STRIPE // BLK):
        pltpu.async_copy(z, acc.at[pl.ds(s * STRIPE + k * BLK, BLK)], sema)
    for k in range(STRIPE // BLK):
        pltpu.make_async_copy(z, acc.at[pl.ds(s * STRIPE, BLK)], sema).wait()
    plsc.subcore_barrier()
    pltpu.sync_copy(row_hbm.at[wid], rowv)
    pltpu.sync_copy(col_hbm.at[wid], colv)
    pltpu.sync_copy(ew_hbm.at[wid], ewv)

    def grp(g, carry):
        base = g * GRP

        @pl.when(g > 0)
        def _drain_prev():
            for j in range(GRP):
                pltpu.make_async_copy(bufs.at[j],
                                      acc.at[colv.at[base + j]], sema).wait()

        for j in range(GRP):
            pltpu.async_copy(hp_hbm.at[rowv.at[base + j]], bufs.at[j], semg)
        for j in range(GRP):
            pltpu.make_async_copy(hp_hbm.at[rowv.at[base + j]],
                                  bufs.at[j], semg).wait()
        for j in range(GRP):
            buf = bufs.at[j]

            def scale16(t, cc, _b=base + j, _buf=buf):
                ew16 = ewv[_b, pl.ds(t * 16, 16)]
                for u in range(16):
                    r = t * 16 + u
                    _buf[r] = _buf[r] * ew16[u]
                return cc

            lax.fori_loop(0, BLK // 16, scale16, 0)
            pltpu.async_copy(buf, acc.at[colv.at[base + j]], sema, add=True)
        return carry

    lax.fori_loop(0, NG, grp, 0)
    for j in range(GRP):
        pltpu.make_async_copy(bufs.at[j],
                              acc.at[colv.at[BPT - GRP + j]], sema).wait()
    plsc.subcore_barrier()
    pltpu.sync_copy(acc.at[pl.ds(s * STRIPE, STRIPE)],
                    out_hbm.at[pl.ds(c * NP + s * STRIPE, STRIPE)])


@functools.partial(
    pl.kernel,
    out_type=jax.ShapeDtypeStruct((NC * NP,), _f32),
    mesh=_mesh,
    scratch_types=[
        pltpu.VMEM((BPT, BLK), jnp.int32),
        pltpu.VMEM((BPT, BLK), jnp.int32),
        pltpu.VMEM((BPT, BLK), _f32),
        pltpu.VMEM((GRP, BLK), _f32),
        pltpu.VMEM_SHARED((NP,), _f32),
        pltpu.SemaphoreType.DMA,
        pltpu.SemaphoreType.DMA,
    ],
    compiler_params=_sc_params,
)
def _sc_msgpass1(row_hbm, col_hbm, ew_hbm, hp_hbm, out_hbm,
                 rowv, colv, ewv, bufs, acc, semg, sema):
    c = lax.axis_index("c")
    s = lax.axis_index("s")
    wid = s * NC + c
    z = bufs.at[0]
    for v in range(BLK // 16):
        z[pl.ds(v * 16, 16)] = jnp.zeros((16,), _f32)
    for k in range(STRIPE // BLK):
        pltpu.async_copy(z, acc.at[pl.ds(s * STRIPE + k * BLK, BLK)], sema)
    for k in range(STRIPE // BLK):
        pltpu.make_async_copy(z, acc.at[pl.ds(s * STRIPE, BLK)], sema).wait()
    plsc.subcore_barrier()
    pltpu.sync_copy(row_hbm.at[wid], rowv)
    pltpu.sync_copy(col_hbm.at[wid], colv)
    pltpu.sync_copy(ew_hbm.at[wid], ewv)

    def grp(g, carry):
        base = g * GRP

        @pl.when(g > 0)
        def _drain_prev():
            for j in range(GRP):
                pltpu.make_async_copy(bufs.at[j],
                                      acc.at[colv.at[base + j]], sema).wait()

        for j in range(GRP):
            pltpu.async_copy(hp_hbm.at[rowv.at[base + j]], bufs.at[j], semg)
        for j in range(GRP):
            pltpu.make_async_copy(hp_hbm.at[rowv.at[base + j]],
                                  bufs.at[j], semg).wait()
        for j in range(GRP):
            buf = bufs.at[j]
            for v in range(BLK // 16):
                sl = pl.ds(v * 16, 16)
                buf[sl] = buf[sl] * ewv[base + j, sl]
            pltpu.async_copy(buf, acc.at[colv.at[base + j]], sema, add=True)
        return carry

    lax.fori_loop(0, NG, grp, 0)
    for j in range(GRP):
        pltpu.make_async_copy(bufs.at[j],
                              acc.at[colv.at[BPT - GRP + j]], sema).wait()
    plsc.subcore_barrier()
    pltpu.sync_copy(acc.at[pl.ds(s * STRIPE, STRIPE)],
                    out_hbm.at[pl.ds(c * NP + s * STRIPE, STRIPE)])


def _dense1_body(d0, d1, x, w1, dis_o, h1p_o):
    dis = lax.rsqrt(d0[...] + d1[...] + 1.0)
    dis_o[...] = dis
    h1 = jnp.dot(x[...], w1[...], preferred_element_type=_f32)
    h1p_o[...] = h1 * dis


def _dense2_body(a0, a1, h1p, dis, b1, w2, h2p_o):
    y1 = jnp.maximum(dis[...] * (a0[...] + a1[...] + h1p[...]) + b1[...], 0.0)
    h2 = jnp.dot(y1, w2[...], preferred_element_type=_f32)
    h2p_o[...] = h2 * dis[...]


def _final_body(q0, q1, h2p, dis, b2, out_o):
    out_o[...] = dis[...] * (q0[...] + q1[...] + h2p[...]) + b2[...]


_dense1 = pl.pallas_call(
    _dense1_body,
    out_shape=[jax.ShapeDtypeStruct((NP, 1), _f32),
               jax.ShapeDtypeStruct((NP, D_HID), _f32)],
)

_dense2 = pl.pallas_call(
    _dense2_body,
    out_shape=jax.ShapeDtypeStruct((NP, 1), _f32),
)

_final = pl.pallas_call(
    _final_body,
    out_shape=jax.ShapeDtypeStruct((NP, 1), _f32),
)


def kernel(x, edge_index, edge_weight, W1, b1, W2, b2):
    ei = edge_index.astype(jnp.int32)
    row = jnp.pad(ei[0], (0, EP - E)).reshape(NW, BPT, BLK)
    col = jnp.pad(ei[1], (0, EP - E)).reshape(NW, BPT, BLK)
    ew = jnp.pad(edge_weight.astype(_f32), (0, EP - E)).reshape(NW, BPT, BLK)
    x_p = jnp.pad(x.astype(_f32), ((0, NP - N), (0, 0)))

    degp = _sc_degree(col, ew)
    dis, h1p = _dense1(degp[:NP].reshape(NP, 1), degp[NP:].reshape(NP, 1),
                       x_p, W1.astype(_f32))
    accp = _sc_msgpass16(row, col, ew, h1p)
    h2p = _dense2(accp[:NP], accp[NP:], h1p, dis,
                  b1.astype(_f32).reshape(1, D_HID), W2.astype(_f32))
    qp = _sc_msgpass1(row, col, ew, h2p.reshape(NP))
    out = _final(qp[:NP].reshape(NP, 1), qp[NP:].reshape(NP, 1),
                 h2p, dis, b2.astype(_f32).reshape(1, 1))
    return out[:N]


# speedup vs baseline: 49.8788x; 1.2687x over previous
import functools

import jax
import jax.numpy as jnp
from jax import lax
from jax.experimental import pallas as pl
from jax.experimental.pallas import tpu as pltpu
from jax.experimental.pallas import tpu_sc as plsc

N = 10000
E = 320000
D_IN = 128
D_HID = 16

NC = 2
NS = 16
NW = NC * NS
BLK = 128
GRP = 8
BPT = 80
NG = BPT // GRP
EPT = BPT * BLK
EP = NW * EPT
NP = 10240
STRIPE = NP // NS

_mesh = plsc.VectorSubcoreMesh(core_axis_name="c", subcore_axis_name="s")
_f32 = jnp.float32
_sc_params = pltpu.CompilerParams(use_tc_tiling_on_sc=False,
                                  needs_layout_passes=False)


@functools.partial(
    pl.kernel,
    out_type=jax.ShapeDtypeStruct((NC * NP,), _f32),
    mesh=_mesh,
    scratch_types=[
        pltpu.VMEM((BPT, BLK), jnp.int32),
        pltpu.VMEM((BPT, BLK), _f32),
        pltpu.VMEM((BLK,), _f32),
        pltpu.VMEM_SHARED((NP,), _f32),
        pltpu.SemaphoreType.DMA,
    ],
    compiler_params=_sc_params,
)
def _sc_degree(col_hbm, ew_hbm, out_hbm, colv, ewv, zbuf, acc, sem):
    c = lax.axis_index("c")
    s = lax.axis_index("s")
    wid = s * NC + c
    for v in range(BLK // 16):
        zbuf[pl.ds(v * 16, 16)] = jnp.zeros((16,), _f32)
    for k in range(STRIPE // BLK):
        pltpu.async_copy(zbuf, acc.at[pl.ds(s * STRIPE + k * BLK, BLK)], sem)
    for k in range(STRIPE // BLK):
        pltpu.make_async_copy(zbuf, acc.at[pl.ds(s * STRIPE, BLK)], sem).wait()
    plsc.subcore_barrier()
    pltpu.sync_copy(col_hbm.at[wid], colv)
    pltpu.sync_copy(ew_hbm.at[wid], ewv)

    def grp(g, carry):
        base = g * GRP
        for j in range(GRP):
            pltpu.async_copy(ewv.at[base + j], acc.at[colv.at[base + j]],
                             sem, add=True)
        for j in range(GRP):
            pltpu.make_async_copy(ewv.at[base + j],
                                  acc.at[colv.at[base + j]], sem).wait()
        return carry

    lax.fori_loop(0, NG, grp, 0)
    plsc.subcore_barrier()
    pltpu.sync_copy(acc.at[pl.ds(s * STRIPE, STRIPE)],
                    out_hbm.at[pl.ds(c * NP + s * STRIPE, STRIPE)])


@functools.partial(
    pl.kernel,
    out_type=jax.ShapeDtypeStruct((NC * NP, D_HID), _f32),
    mesh=_mesh,
    scratch_types=[
        pltpu.VMEM((BPT, BLK), jnp.int32),
        pltpu.VMEM((BPT, BLK), jnp.int32),
        pltpu.VMEM((BPT, BLK), _f32),
        pltpu.VMEM((GRP, BLK, D_HID), _f32),
        pltpu.VMEM_SHARED((NP, D_HID), _f32),
        pltpu.SemaphoreType.DMA,
        pltpu.SemaphoreType.DMA,
    ],
    compiler_params=_sc_params,
)
def _sc_msgpass16(row_hbm, col_hbm, ew_hbm, hp_hbm, out_hbm,
                  rowv, colv, ewv, bufs, acc, semg, sema):
    c = lax.axis_index("c")
    s = lax.axis_index("s")
    wid = s * NC + c
    z = bufs.at[0]
    zero16 = jnp.zeros((16,), _f32)

    def zi(j, carry):
        z[j] = zero16
        return carry

    lax.fori_loop(0, BLK, zi, 0)
    for k in range(STRIPE // BLK):
        pltpu.async_copy(z, acc.at[pl.ds(s * STRIPE + k * BLK, BLK)], sema)
    for k in range(STRIPE // BLK):
        pltpu.make_async_copy(z, acc.at[pl.ds(s * STRIPE, BLK)], sema).wait()
    plsc.subcore_barrier()
    pltpu.sync_copy(row_hbm.at[wid], rowv)
    pltpu.sync_copy(col_hbm.at[wid], colv)
    pltpu.sync_copy(ew_hbm.at[wid], ewv)

    def grp(g, carry):
        base = g * GRP

        @pl.when(g > 0)
        def _drain_prev():
            for j in range(GRP):
                pltpu.make_async_copy(bufs.at[j],
                                      acc.at[colv.at[base + j]], sema).wait()

        for j in range(GRP):
            pltpu.async_copy(hp_hbm.at[rowv.at[base + j]], bufs.at[j], semg)
        for j in range(GRP):
            pltpu.make_async_copy(hp_hbm.at[rowv.at[base + j]],
                                  bufs.at[j], semg).wait()
        for j in range(GRP):
            buf = bufs.at[j]

            def scale16(t, cc, _b=base + j, _buf=buf):
                ew16 = ewv[_b, pl.ds(t * 16, 16)]
                for u in range(16):
                    r = t * 16 + u
                    _buf[r] = _buf[r] * ew16[u]
                return cc

            lax.fori_loop(0, BLK // 16, scale16, 0)
            pltpu.async_copy(buf, acc.at[colv.at[base + j]], sema, add=True)
        return carry

    lax.fori_loop(0, NG, grp, 0)
    for j in range(GRP):
        pltpu.make_async_copy(bufs.at[j],
                              acc.at[colv.at[BPT - GRP + j]], sema).wait()
    plsc.subcore_barrier()
    pltpu.sync_copy(acc.at[pl.ds(s * STRIPE, STRIPE)],
                    out_hbm.at[pl.ds(c * NP + s * STRIPE, STRIPE)])


@functools.partial(
    pl.kernel,
    out_type=jax.ShapeDtypeStruct((NC * NP,), _f32),
    mesh=_mesh,
    scratch_types=[
        pltpu.VMEM((BPT, BLK), jnp.int32),
        pltpu.VMEM((BPT, BLK), jnp.int32),
        pltpu.VMEM((BPT, BLK), _f32),
        pltpu.VMEM((NP,), _f32),
        pltpu.VMEM((GRP, BLK), _f32),
        pltpu.VMEM_SHARED((NP,), _f32),
        pltpu.SemaphoreType.DMA,
    ],
    compiler_params=_sc_params,
)
def _sc_msgpass1(row_hbm, col_hbm, ew_hbm, hp_hbm, out_hbm,
                 rowv, colv, ewv, table, bufs, acc, sema):
    c = lax.axis_index("c")
    s = lax.axis_index("s")
    wid = s * NC + c
    z = bufs.at[0]
    for v in range(BLK // 16):
        z[pl.ds(v * 16, 16)] = jnp.zeros((16,), _f32)
    for k in range(STRIPE // BLK):
        pltpu.async_copy(z, acc.at[pl.ds(s * STRIPE + k * BLK, BLK)], sema)
    for k in range(STRIPE // BLK):
        pltpu.make_async_copy(z, acc.at[pl.ds(s * STRIPE, BLK)], sema).wait()
    plsc.subcore_barrier()
    pltpu.sync_copy(row_hbm.at[wid], rowv)
    pltpu.sync_copy(col_hbm.at[wid], colv)
    pltpu.sync_copy(ew_hbm.at[wid], ewv)
    pltpu.sync_copy(hp_hbm, table)

    def grp(g, carry):
        base = g * GRP

        @pl.when(g > 0)
        def _drain_prev():
            for j in range(GRP):
                pltpu.make_async_copy(bufs.at[j],
                                      acc.at[colv.at[base + j]], sema).wait()

        for j in range(GRP):
            b = base + j
            buf = bufs.at[j]
            for v in range(BLK // 16):
                sl = pl.ds(v * 16, 16)
                vals = plsc.load_gather(table, [rowv[b, sl]])
                buf[sl] = vals * ewv[b, sl]
            pltpu.async_copy(buf, acc.at[colv.at[b]], sema, add=True)
        return carry

    lax.fori_loop(0, NG, grp, 0)
    for j in range(GRP):
        pltpu.make_async_copy(bufs.at[j],
                              acc.at[colv.at[BPT - GRP + j]], sema).wait()
    plsc.subcore_barrier()
    pltpu.sync_copy(acc.at[pl.ds(s * STRIPE, STRIPE)],
                    out_hbm.at[pl.ds(c * NP + s * STRIPE, STRIPE)])


def _dense1_body(d0, d1, x, w1, dis_o, h1p_o):
    dis = lax.rsqrt(d0[...] + d1[...] + 1.0)
    dis_o[...] = dis
    h1 = jnp.dot(x[...], w1[...], preferred_element_type=_f32)
    h1p_o[...] = h1 * dis


def _dense2_body(a0, a1, h1p, dis, b1, w2, h2p_o):
    y1 = jnp.maximum(dis[...] * (a0[...] + a1[...] + h1p[...]) + b1[...], 0.0)
    h2 = jnp.dot(y1, w2[...], preferred_element_type=_f32)
    h2p_o[...] = h2 * dis[...]


def _final_body(q0, q1, h2p, dis, b2, out_o):
    out_o[...] = dis[...] * (q0[...] + q1[...] + h2p[...]) + b2[...]


_dense1 = pl.pallas_call(
    _dense1_body,
    out_shape=[jax.ShapeDtypeStruct((NP, 1), _f32),
               jax.ShapeDtypeStruct((NP, D_HID), _f32)],
)

_dense2 = pl.pallas_call(
    _dense2_body,
    out_shape=jax.ShapeDtypeStruct((NP, 1), _f32),
)

_final = pl.pallas_call(
    _final_body,
    out_shape=jax.ShapeDtypeStruct((NP, 1), _f32),
)


def kernel(x, edge_index, edge_weight, W1, b1, W2, b2):
    ei = edge_index.astype(jnp.int32)
    row = jnp.pad(ei[0], (0, EP - E)).reshape(NW, BPT, BLK)
    col = jnp.pad(ei[1], (0, EP - E)).reshape(NW, BPT, BLK)
    ew = jnp.pad(edge_weight.astype(_f32), (0, EP - E)).reshape(NW, BPT, BLK)
    x_p = jnp.pad(x.astype(_f32), ((0, NP - N), (0, 0)))

    degp = _sc_degree(col, ew)
    dis, h1p = _dense1(degp[:NP].reshape(NP, 1), degp[NP:].reshape(NP, 1),
                       x_p, W1.astype(_f32))
    accp = _sc_msgpass16(row, col, ew, h1p)
    h2p = _dense2(accp[:NP], accp[NP:], h1p, dis,
                  b1.astype(_f32).reshape(1, D_HID), W2.astype(_f32))
    qp = _sc_msgpass1(row, col, ew, h2p.reshape(NP))
    out = _final(qp[:NP].reshape(NP, 1), qp[NP:].reshape(NP, 1),
                 h2p, dis, b2.astype(_f32).reshape(1, 1))
    return out[:N]
